# baseline (speedup 1.0000x reference)
import jax
import jax.numpy as jnp
from jax.experimental import pallas as pl
from jax.experimental.pallas import tpu as pltpu

B, T, IW, E = 32, 8, 32, 8
D = IW * IW
HID = 2 * D
NM = 5
f32 = jnp.float32
bf16 = jnp.bfloat16


def _bf(a):
    return a.astype(bf16).astype(f32)


def _cv2(v):
    m = jnp.mean(v)
    var = jnp.sum((v - m) ** 2) / (E - 1)
    return var / (m * m + 1e-10)


def _stem_kernel(xf_ref, w1_ref, b1c_ref, w2s_ref, b2c_ref, x_ref):
    xf = _bf(xf_ref[0])
    lane = jax.lax.broadcasted_iota(jnp.int32, (1, 1024), 1)
    hh = lane // IW
    ww = lane % IW
    dd = jax.lax.broadcasted_iota(jnp.int32, (T, 1), 0)

    def shift(a, sd, sh, sw):
        if sh or sw:
            a = jnp.roll(a, -(sh * IW + sw), axis=1)
            cond = jnp.ones((1, 1024), jnp.bool_)
            if sh:
                cond = cond & ((hh + sh >= 0) & (hh + sh < IW))
            if sw:
                cond = cond & ((ww + sw >= 0) & (ww + sw < IW))
            a = jnp.where(cond, a, 0.0)
        if sd:
            a = jnp.roll(a, -sd, axis=0)
            condr = (dd + sd >= 0) & (dd + sd < T)
            a = jnp.where(condr, a, 0.0)
        return a

    acc = [None] * 10
    k = 0
    for kd in range(3):
        for kh in range(3):
            for kw in range(3):
                s = shift(xf, kd - 1, kh - 1, kw - 1)
                for c in range(10):
                    t = s * _bf(w1_ref[c, k])
                    acc[c] = t if acc[c] is None else acc[c] + t
                k += 1
    out1 = [_bf(jnp.maximum(acc[c] + b1c_ref[0, c], 0.0)) for c in range(10)]

    acc2 = None
    for kh in range(3):
        for kw in range(3):
            j = kh * 3 + kw
            for c in range(10):
                wcol = jnp.zeros((T, 1), f32)
                for d in range(T):
                    wcol = jnp.where(dd == d, w2s_ref[c, d * 9 + j], wcol)
                t = shift(out1[c], 0, kh - 1, kw - 1) * wcol
                acc2 = t if acc2 is None else acc2 + t
    xs = acc2.sum(axis=0, keepdims=True) + b2c_ref[0, 0]
    x_ref[0] = jnp.maximum(xs, 0.0)


def _ffn_kernel(x_ref, w1_ref, b1_ref, w2_ref, b2_ref, p_ref):
    xb = x_ref[...].astype(bf16)
    h = jnp.dot(xb, w1_ref[0, 0].astype(bf16),
                preferred_element_type=f32) + b1_ref[0, 0]
    h = jnp.maximum(h, 0.0).astype(bf16)
    o = jnp.dot(h, w2_ref[0, 0].astype(bf16),
                preferred_element_type=f32) + b2_ref[0, 0]
    m = jnp.max(o, axis=1, keepdims=True)
    p = jnp.exp(o - m)
    p_ref[0, 0] = p / jnp.sum(p, axis=1, keepdims=True)


def _combine_kernel(inp_ref, p_ref, x_ref, wg_ref, out_ref, aux_ref):
    lg_all = jnp.dot(x_ref[...].astype(bf16), wg_ref[...].astype(bf16),
                     preferred_element_type=f32)
    ei = jax.lax.broadcasted_iota(jnp.int32, (B, E), 1)
    aux = jnp.float32(0.0)
    funcs = []
    for i in range(NM):
        lg = lg_all[:, i * E:(i + 1) * E]
        m1 = jnp.max(lg, axis=1, keepdims=True)
        idx1 = jnp.min(jnp.where(lg == m1, ei, E), axis=1, keepdims=True)
        oh1 = ei == idx1
        masked = jnp.where(oh1, -jnp.inf, lg)
        m2 = jnp.max(masked, axis=1, keepdims=True)
        idx2 = jnp.min(jnp.where(masked == m2, ei, E), axis=1, keepdims=True)
        oh2 = ei == idx2
        e2 = jnp.exp(m2 - m1)
        g1 = 1.0 / (1.0 + e2)
        g2 = e2 / (1.0 + e2)
        gates = jnp.where(oh1, g1, 0.0) + jnp.where(oh2, g2, 0.0)
        imp = jnp.sum(gates, axis=0, keepdims=True)
        load = jnp.sum((gates > 0).astype(f32), axis=0, keepdims=True)
        aux = aux + (_cv2(imp) + _cv2(load)) * 1e-2
        accf = None
        for e in range(E):
            t = p_ref[i, e] * gates[:, e:e + 1]
            accf = t if accf is None else accf + t
        funcs.append(accf[:, None, :])
    aux_ref[0, 0] = aux
    f1, f0, f2, f3, f4 = funcs
    x = inp_ref[...]
    x2 = x * x
    x3 = x2 * x
    x4 = x2 * x2
    arg = x4 * f4 + x3 * f3 + x2 * f2 + x * f1 + f0
    out_ref[...] = 1.0 / (1.0 + jnp.exp(-arg))


def kernel(input, conv1_w, conv1_b, conv2_w, conv2_b, w_gate, W1, b1, W2, b2):
    xf = input.reshape(B, T, 1024)
    w1f = conv1_w.reshape(10, 27)
    b1c = conv1_b.reshape(1, 10)
    w2s = conv2_w.reshape(10, T * 9)
    b2c = conv2_b.reshape(1, 1)
    wgf = jnp.transpose(w_gate, (1, 0, 2)).reshape(1024, NM * E)

    smem = pl.BlockSpec(memory_space=pltpu.SMEM)
    x3 = pl.pallas_call(
        _stem_kernel,
        grid=(B,),
        in_specs=[pl.BlockSpec((1, T, 1024), lambda b: (b, 0, 0)),
                  smem, smem, smem, smem],
        out_specs=pl.BlockSpec((1, 1, 1024), lambda b: (b, 0, 0)),
        out_shape=jax.ShapeDtypeStruct((B, 1, 1024), f32),
        compiler_params=pltpu.CompilerParams(
            dimension_semantics=("arbitrary",)),
    )(xf, w1f, b1c, w2s, b2c)
    x = x3.reshape(B, 1024)

    b1r = b1.reshape(NM, E, 1, HID)
    b2r = b2.reshape(NM, E, 1, D)
    p = pl.pallas_call(
        _ffn_kernel,
        grid=(E, NM),
        in_specs=[pl.BlockSpec((B, 1024), lambda e, i: (0, 0)),
                  pl.BlockSpec((1, 1, 1024, HID), lambda e, i: (i, e, 0, 0)),
                  pl.BlockSpec((1, 1, 1, HID), lambda e, i: (i, e, 0, 0)),
                  pl.BlockSpec((1, 1, HID, D), lambda e, i: (i, e, 0, 0)),
                  pl.BlockSpec((1, 1, 1, D), lambda e, i: (i, e, 0, 0))],
        out_specs=pl.BlockSpec((1, 1, B, D), lambda e, i: (i, e, 0, 0)),
        out_shape=jax.ShapeDtypeStruct((NM, E, B, D), f32),
        compiler_params=pltpu.CompilerParams(
            dimension_semantics=("parallel", "parallel")),
    )(x, W1, b1r, W2, b2r)

    inp3 = input.reshape(B, T, 1024)
    out, aux = pl.pallas_call(
        _combine_kernel,
        in_specs=[pl.BlockSpec(inp3.shape, lambda: (0, 0, 0)),
                  pl.BlockSpec(p.shape, lambda: (0, 0, 0, 0)),
                  pl.BlockSpec((B, 1024), lambda: (0, 0)),
                  pl.BlockSpec(wgf.shape, lambda: (0, 0))],
        out_specs=[pl.BlockSpec(inp3.shape, lambda: (0, 0, 0)),
                   pl.BlockSpec(memory_space=pltpu.SMEM)],
        out_shape=[jax.ShapeDtypeStruct(inp3.shape, f32),
                   jax.ShapeDtypeStruct((1, 1), f32)],
    )(inp3, p, x, wgf)

    return out.reshape(B, T, 1, IW, IW), aux.reshape(())
